# baseline (speedup 1.0000x reference)
import numpy as np
import jax
import jax.numpy as jnp
from jax.experimental import pallas as pl
from jax.experimental.pallas import tpu as pltpu

N = 16384
IN = 4
NF = 6
FEAT = 256
PH = 128
E = 8
EH = 256
OUT = 1
FUSED = FEAT + PH

TILE = 512
TE = 256
M_PAD = 2 * N + E * TE
NT = M_PAD // TE

OMEGA = 30.0

_PIO2 = np.pi / 2
_C1 = np.float32(np.floor(_PIO2 * 2 ** 11) / 2 ** 11)
_C2 = np.float32(np.floor((_PIO2 - float(_C1)) * 2 ** 28) / 2 ** 28)
_C3 = np.float32(_PIO2 - float(_C1) - float(_C2))
_TWO_OVER_PI = np.float32(2.0 / np.pi)
_MAGIC = np.float32(1.5 * 2 ** 23)
_MAGIC_I = np.int32(0x4B400000)
_S1 = np.float32(-1.66666546e-1)
_S2 = np.float32(8.33216087e-3)
_S3 = np.float32(-1.95152959e-4)
_K1 = np.float32(-0.5)
_K2 = np.float32(4.16666418e-2)
_K3 = np.float32(-1.38873162e-3)
_K4 = np.float32(2.44331571e-5)


def _fast_sin(x):
    qf = x * _TWO_OVER_PI + _MAGIC
    qi = jax.lax.bitcast_convert_type(qf, jnp.int32) - _MAGIC_I
    q = qi.astype(jnp.float32)
    r = x - q * _C1
    r = r - q * _C2
    r = r - q * _C3
    r2 = r * r
    sp = r + r * (r2 * (_S1 + r2 * (_S2 + r2 * _S3)))
    cp = 1.0 + r2 * (_K1 + r2 * (_K2 + r2 * (_K3 + r2 * _K4)))
    res = jnp.where((qi & 1) == 1, cp, sp)
    return jnp.where((qi & 2) == 2, -res, res)


def _front_kernel(x_ref, fr_ref, w1_ref, eb1_ref, ew2_ref, eb2_ref,
                  pw1_ref, pb1_ref, pw2_ref, pb2_ref, pw3_ref, pb3_ref,
                  pwh_ref, pbh_ref,
                  fused_ref, eidx_ref, g_ref):
    x = x_ref[...]
    f32 = jnp.float32

    xr = jnp.repeat(x, NF, axis=1) * fr_ref[...]
    enc = jnp.concatenate([x, jnp.sin(xr), jnp.cos(xr)], axis=1)

    h = jnp.sin(OMEGA * (jnp.dot(enc, w1_ref[...],
                                 preferred_element_type=f32) + eb1_ref[...]))
    feat = jnp.sin(OMEGA * (jnp.dot(h, ew2_ref[...],
                                    preferred_element_type=f32) + eb2_ref[...]))

    p = jnp.sin(OMEGA * (jnp.dot(x, pw1_ref[...],
                                 preferred_element_type=f32) + pb1_ref[...]))
    p = jnp.sin(OMEGA * (jnp.dot(p, pw2_ref[...],
                                 preferred_element_type=f32) + pb2_ref[...]))
    p = jnp.sin(OMEGA * (jnp.dot(p, pw3_ref[...],
                                 preferred_element_type=f32) + pb3_ref[...]))
    logits = jnp.dot(p, pwh_ref[...], preferred_element_type=f32) + pbh_ref[...]

    iota = jax.lax.broadcasted_iota(jnp.int32, logits.shape, 1)
    m1 = jnp.max(logits, axis=-1, keepdims=True)
    idx1 = jnp.min(jnp.where(logits >= m1, iota, E), axis=-1, keepdims=True)
    mask1 = iota == idx1
    rest = jnp.where(mask1, -jnp.inf, logits)
    m2 = jnp.max(rest, axis=-1, keepdims=True)
    idx2 = jnp.min(jnp.where(rest >= m2, iota, E), axis=-1, keepdims=True)
    e2 = jnp.exp(m2 - m1)
    g1 = 1.0 / (1.0 + e2)
    g2 = e2 / (1.0 + e2)

    fused_ref[...] = jnp.concatenate([feat, p], axis=1)
    eidx_ref[...] = jnp.concatenate([idx1, idx2], axis=1)
    g_ref[...] = jnp.concatenate([g1, g2], axis=1)


def _expert_kernel(te_ref, rows_ref, w0_ref, b0_ref, w1_ref, b1_ref,
                   w2_ref, b2_ref, wo_ref, bo_ref, y_ref):
    f32 = jnp.float32
    r = rows_ref[...]
    h0 = _fast_sin(OMEGA * (jnp.dot(r, w0_ref[0],
                                    preferred_element_type=f32) + b0_ref[0]))
    h1 = _fast_sin(OMEGA * (jnp.dot(h0, w1_ref[0],
                                    preferred_element_type=f32) + b1_ref[0]))
    h2 = _fast_sin(OMEGA * (jnp.dot(h1, w2_ref[0],
                                    preferred_element_type=f32) + b2_ref[0]))
    y = jnp.dot(h2, wo_ref[0], preferred_element_type=f32)
    y_ref[...] = y + bo_ref[0][:, :1]


def _combine_kernel(g_ref, y1_ref, y2_ref, o_ref):
    g = g_ref[...]
    o_ref[...] = g[:, :1] * y1_ref[...] + g[:, 1:2] * y2_ref[...]


def kernel(x, enc_W1, enc_b1, enc_W2, enc_b2,
           pol_W1, pol_b1, pol_W2, pol_b2, pol_W3, pol_b3, pol_Wh, pol_bh,
           exp_W0, exp_b0, exp_W1, exp_b1, exp_W2, exp_b2, exp_Wo, exp_bo):
    f32 = jnp.float32
    freqs = (2.0 ** np.arange(NF, dtype=np.float32)) * np.pi
    fr = jnp.asarray(np.tile(freqs, IN).reshape(1, IN * NF))

    def row2(a):
        return a.reshape(1, -1).astype(f32)

    bf = lambda a: a.astype(jnp.bfloat16)

    full = lambda shape: pl.BlockSpec(shape, lambda i: (0,) * len(shape))

    fused, eidx, g = pl.pallas_call(
        _front_kernel,
        grid=(N // TILE,),
        in_specs=[
            pl.BlockSpec((TILE, IN), lambda i: (i, 0)),
            full(fr.shape), full(enc_W1.shape),
            full((1, FEAT)), full(enc_W2.shape), full((1, FEAT)),
            full(pol_W1.shape), full((1, PH)), full(pol_W2.shape),
            full((1, PH)), full(pol_W3.shape), full((1, PH)),
            full(pol_Wh.shape), full((1, E)),
        ],
        out_specs=[pl.BlockSpec((TILE, FUSED), lambda i: (i, 0)),
                   pl.BlockSpec((TILE, 2), lambda i: (i, 0)),
                   pl.BlockSpec((TILE, 2), lambda i: (i, 0))],
        out_shape=[jax.ShapeDtypeStruct((N, FUSED), f32),
                   jax.ShapeDtypeStruct((N, 2), jnp.int32),
                   jax.ShapeDtypeStruct((N, 2), f32)],
        compiler_params=pltpu.CompilerParams(
            dimension_semantics=("arbitrary",)),
    )(x, fr, bf(enc_W1), row2(enc_b1), bf(enc_W2), row2(enc_b2),
      bf(pol_W1), row2(pol_b1), bf(pol_W2), row2(pol_b2),
      bf(pol_W3), row2(pol_b3), bf(pol_Wh), row2(pol_bh))

    keys = jnp.concatenate([eidx[:, 0], eidx[:, 1]])
    order = jnp.argsort(keys, stable=True)
    skeys = keys[order]
    counts = jnp.bincount(keys, length=E)
    pc = ((counts + TE - 1) // TE) * TE
    ends_pad = jnp.cumsum(pc)
    starts_pad = ends_pad - pc
    starts_raw = jnp.cumsum(counts) - counts
    i2n = jnp.arange(2 * N, dtype=jnp.int32)
    dest = (starts_pad[skeys] + (i2n - starts_raw[skeys])).astype(jnp.int32)
    tok_sorted = (order % N).astype(jnp.int32)
    tok_pad = jnp.zeros((M_PAD,), jnp.int32).at[dest].set(tok_sorted)
    rows = fused[tok_pad]
    tile_expert = jnp.clip(
        jnp.searchsorted(ends_pad, jnp.arange(NT) * TE, side="right"),
        0, E - 1).astype(jnp.int32)
    dest_unsorted = jnp.zeros((2 * N,), jnp.int32).at[order].set(dest)
    pos1, pos2 = dest_unsorted[:N], dest_unsorted[N:]

    bo128 = jnp.broadcast_to(exp_bo, (E, 128))
    grid_spec = pltpu.PrefetchScalarGridSpec(
        num_scalar_prefetch=1,
        grid=(NT,),
        in_specs=[
            pl.BlockSpec((TE, FUSED), lambda t, te: (t, 0)),
            pl.BlockSpec((1, FUSED, EH), lambda t, te: (te[t], 0, 0)),
            pl.BlockSpec((1, 1, EH), lambda t, te: (te[t], 0, 0)),
            pl.BlockSpec((1, EH, EH), lambda t, te: (te[t], 0, 0)),
            pl.BlockSpec((1, 1, EH), lambda t, te: (te[t], 0, 0)),
            pl.BlockSpec((1, EH, EH), lambda t, te: (te[t], 0, 0)),
            pl.BlockSpec((1, 1, EH), lambda t, te: (te[t], 0, 0)),
            pl.BlockSpec((1, EH, OUT), lambda t, te: (te[t], 0, 0)),
            pl.BlockSpec((1, 1, 128), lambda t, te: (te[t], 0, 0)),
        ],
        out_specs=pl.BlockSpec((TE, OUT), lambda t, te: (t, 0)),
    )
    y_sorted = pl.pallas_call(
        _expert_kernel,
        grid_spec=grid_spec,
        out_shape=jax.ShapeDtypeStruct((M_PAD, OUT), f32),
        compiler_params=pltpu.CompilerParams(
            dimension_semantics=("arbitrary",)),
    )(tile_expert, rows, bf(exp_W0), exp_b0[:, None, :], bf(exp_W1),
      exp_b1[:, None, :], bf(exp_W2), exp_b2[:, None, :], bf(exp_Wo),
      bo128[:, None, :])

    y1 = y_sorted[pos1]
    y2 = y_sorted[pos2]
    out = pl.pallas_call(
        _combine_kernel,
        grid=(N // TILE,),
        in_specs=[pl.BlockSpec((TILE, 2), lambda i: (i, 0)),
                  pl.BlockSpec((TILE, OUT), lambda i: (i, 0)),
                  pl.BlockSpec((TILE, OUT), lambda i: (i, 0))],
        out_specs=pl.BlockSpec((TILE, OUT), lambda i: (i, 0)),
        out_shape=jax.ShapeDtypeStruct((N, OUT), f32),
        compiler_params=pltpu.CompilerParams(
            dimension_semantics=("arbitrary",)),
    )(g, y1, y2)
    return out


# speedup vs baseline: 1.1747x; 1.1747x over previous
import numpy as np
import jax
import jax.numpy as jnp
from jax.experimental import pallas as pl
from jax.experimental.pallas import tpu as pltpu

N = 16384
IN = 4
NF = 6
FEAT = 256
PH = 128
E = 8
EH = 256
OUT = 1
FUSED = FEAT + PH

TILE = 512
TE = 256
M_PAD = 2 * N + E * TE
NT = M_PAD // TE

OMEGA = 30.0

_PIO2 = np.pi / 2
_C1 = np.float32(np.floor(_PIO2 * 2 ** 11) / 2 ** 11)
_C2 = np.float32(np.floor((_PIO2 - float(_C1)) * 2 ** 28) / 2 ** 28)
_C3 = np.float32(_PIO2 - float(_C1) - float(_C2))
_TWO_OVER_PI = np.float32(2.0 / np.pi)
_MAGIC = np.float32(1.5 * 2 ** 23)
_MAGIC_I = np.int32(0x4B400000)
_S1 = np.float32(-1.66666546e-1)
_S2 = np.float32(8.33216087e-3)
_S3 = np.float32(-1.95152959e-4)
_K1 = np.float32(-0.5)
_K2 = np.float32(4.16666418e-2)
_K3 = np.float32(-1.38873162e-3)
_K4 = np.float32(2.44331571e-5)


def _fast_sin(x):
    qf = x * _TWO_OVER_PI + _MAGIC
    qi = jax.lax.bitcast_convert_type(qf, jnp.int32) - _MAGIC_I
    q = qi.astype(jnp.float32)
    r = x - q * _C1
    r = r - q * _C2
    r = r - q * _C3
    r2 = r * r
    sp = r + r * (r2 * (_S1 + r2 * (_S2 + r2 * _S3)))
    cp = 1.0 + r2 * (_K1 + r2 * (_K2 + r2 * (_K3 + r2 * _K4)))
    res = jnp.where((qi & 1) == 1, cp, sp)
    return jnp.where((qi & 2) == 2, -res, res)


def _front_kernel(x_ref, fr_ref, tri_ref, w1_ref, eb1_ref, ew2_ref, eb2_ref,
                  pw1_ref, pb1_ref, pw2_ref, pb2_ref, pw3_ref, pb3_ref,
                  pwh_ref, pbh_ref,
                  fused_ref, eidx_ref, g_ref, dloc_ref, counts_ref,
                  carry_ref):
    x = x_ref[...]
    f32 = jnp.float32

    xr = jnp.repeat(x, NF, axis=1) * fr_ref[...]
    enc = jnp.concatenate([x, jnp.sin(xr), jnp.cos(xr)], axis=1)

    h = jnp.sin(OMEGA * (jnp.dot(enc, w1_ref[...],
                                 preferred_element_type=f32) + eb1_ref[...]))
    feat = jnp.sin(OMEGA * (jnp.dot(h, ew2_ref[...],
                                    preferred_element_type=f32) + eb2_ref[...]))

    p = jnp.sin(OMEGA * (jnp.dot(x, pw1_ref[...],
                                 preferred_element_type=f32) + pb1_ref[...]))
    p = jnp.sin(OMEGA * (jnp.dot(p, pw2_ref[...],
                                 preferred_element_type=f32) + pb2_ref[...]))
    p = jnp.sin(OMEGA * (jnp.dot(p, pw3_ref[...],
                                 preferred_element_type=f32) + pb3_ref[...]))
    logits = jnp.dot(p, pwh_ref[...], preferred_element_type=f32) + pbh_ref[...]

    iota = jax.lax.broadcasted_iota(jnp.int32, logits.shape, 1)
    m1 = jnp.max(logits, axis=-1, keepdims=True)
    idx1 = jnp.min(jnp.where(logits >= m1, iota, E), axis=-1, keepdims=True)
    mask1 = iota == idx1
    rest = jnp.where(mask1, -jnp.inf, logits)
    m2 = jnp.max(rest, axis=-1, keepdims=True)
    idx2 = jnp.min(jnp.where(rest >= m2, iota, E), axis=-1, keepdims=True)
    e2 = jnp.exp(m2 - m1)
    g1 = 1.0 / (1.0 + e2)
    g2 = e2 / (1.0 + e2)

    fused_ref[...] = jnp.concatenate([feat, p], axis=1)
    eidx_ref[...] = jnp.concatenate([idx1, idx2], axis=1)
    g_ref[...] = jnp.concatenate([g1, g2], axis=1)

    @pl.when(pl.program_id(0) == 0)
    def _init():
        carry_ref[...] = jnp.zeros_like(carry_ref)

    carry = carry_ref[...]
    oh1 = mask1.astype(f32)
    oh2 = (iota == idx2).astype(f32)
    tri = tri_ref[...]
    excl1 = jnp.dot(tri, oh1, preferred_element_type=f32)
    excl2 = jnp.dot(tri, oh2, preferred_element_type=f32)
    cs1 = jnp.sum(oh1, axis=0, keepdims=True)
    cs2 = jnp.sum(oh2, axis=0, keepdims=True)
    rank1 = jnp.sum(oh1 * (carry + excl1), axis=1, keepdims=True)
    rank2 = jnp.sum(oh2 * (carry + cs1 + excl2), axis=1, keepdims=True)
    dloc_ref[...] = jnp.concatenate([rank1, rank2], axis=1).astype(jnp.int32)
    new_carry = carry + cs1 + cs2
    carry_ref[...] = new_carry
    counts_ref[...] = new_carry


def _expert_kernel(te_ref, rows_ref, w0_ref, b0_ref, w1_ref, b1_ref,
                   w2_ref, b2_ref, wo_ref, bo_ref, y_ref):
    f32 = jnp.float32
    r = rows_ref[...]
    h0 = _fast_sin(OMEGA * (jnp.dot(r, w0_ref[0],
                                    preferred_element_type=f32) + b0_ref[0]))
    h1 = _fast_sin(OMEGA * (jnp.dot(h0, w1_ref[0],
                                    preferred_element_type=f32) + b1_ref[0]))
    h2 = _fast_sin(OMEGA * (jnp.dot(h1, w2_ref[0],
                                    preferred_element_type=f32) + b2_ref[0]))
    y = jnp.dot(h2, wo_ref[0], preferred_element_type=f32)
    y_ref[...] = y + bo_ref[0][:, :1]


def _combine_kernel(g_ref, y1_ref, y2_ref, o_ref):
    g = g_ref[...]
    o_ref[...] = g[:, :1] * y1_ref[...] + g[:, 1:2] * y2_ref[...]


def kernel(x, enc_W1, enc_b1, enc_W2, enc_b2,
           pol_W1, pol_b1, pol_W2, pol_b2, pol_W3, pol_b3, pol_Wh, pol_bh,
           exp_W0, exp_b0, exp_W1, exp_b1, exp_W2, exp_b2, exp_Wo, exp_bo):
    f32 = jnp.float32
    freqs = (2.0 ** np.arange(NF, dtype=np.float32)) * np.pi
    fr = jnp.asarray(np.tile(freqs, IN).reshape(1, IN * NF))

    def row2(a):
        return a.reshape(1, -1).astype(f32)

    bf = lambda a: a.astype(jnp.bfloat16)

    full = lambda shape: pl.BlockSpec(shape, lambda i: (0,) * len(shape))

    tri = jnp.asarray(np.tril(np.ones((TILE, TILE), np.float32), -1))

    fused, eidx, g, dloc, counts_out = pl.pallas_call(
        _front_kernel,
        grid=(N // TILE,),
        in_specs=[
            pl.BlockSpec((TILE, IN), lambda i: (i, 0)),
            full(fr.shape), full(tri.shape), full(enc_W1.shape),
            full((1, FEAT)), full(enc_W2.shape), full((1, FEAT)),
            full(pol_W1.shape), full((1, PH)), full(pol_W2.shape),
            full((1, PH)), full(pol_W3.shape), full((1, PH)),
            full(pol_Wh.shape), full((1, E)),
        ],
        out_specs=[pl.BlockSpec((TILE, FUSED), lambda i: (i, 0)),
                   pl.BlockSpec((TILE, 2), lambda i: (i, 0)),
                   pl.BlockSpec((TILE, 2), lambda i: (i, 0)),
                   pl.BlockSpec((TILE, 2), lambda i: (i, 0)),
                   pl.BlockSpec((1, E), lambda i: (0, 0))],
        out_shape=[jax.ShapeDtypeStruct((N, FUSED), f32),
                   jax.ShapeDtypeStruct((N, 2), jnp.int32),
                   jax.ShapeDtypeStruct((N, 2), f32),
                   jax.ShapeDtypeStruct((N, 2), jnp.int32),
                   jax.ShapeDtypeStruct((1, E), f32)],
        scratch_shapes=[pltpu.VMEM((1, E), f32)],
        compiler_params=pltpu.CompilerParams(
            dimension_semantics=("arbitrary",)),
    )(x, fr, bf(tri), bf(enc_W1), row2(enc_b1), bf(enc_W2), row2(enc_b2),
      bf(pol_W1), row2(pol_b1), bf(pol_W2), row2(pol_b2),
      bf(pol_W3), row2(pol_b3), bf(pol_Wh), row2(pol_bh))

    counts = counts_out[0].astype(jnp.int32)
    pc = ((counts + TE - 1) // TE) * TE
    ends_pad = jnp.cumsum(pc)
    starts_pad = ends_pad - pc
    dest = starts_pad[eidx] + dloc
    tok = jnp.arange(N, dtype=jnp.int32)
    tok_pad = (jnp.zeros((M_PAD,), jnp.int32)
               .at[dest[:, 0]].set(tok).at[dest[:, 1]].set(tok))
    rows = fused[tok_pad]
    tile_expert = jnp.clip(
        jnp.searchsorted(ends_pad, jnp.arange(NT) * TE, side="right"),
        0, E - 1).astype(jnp.int32)
    pos1, pos2 = dest[:, 0], dest[:, 1]

    bo128 = jnp.broadcast_to(exp_bo, (E, 128))
    grid_spec = pltpu.PrefetchScalarGridSpec(
        num_scalar_prefetch=1,
        grid=(NT,),
        in_specs=[
            pl.BlockSpec((TE, FUSED), lambda t, te: (t, 0)),
            pl.BlockSpec((1, FUSED, EH), lambda t, te: (te[t], 0, 0)),
            pl.BlockSpec((1, 1, EH), lambda t, te: (te[t], 0, 0)),
            pl.BlockSpec((1, EH, EH), lambda t, te: (te[t], 0, 0)),
            pl.BlockSpec((1, 1, EH), lambda t, te: (te[t], 0, 0)),
            pl.BlockSpec((1, EH, EH), lambda t, te: (te[t], 0, 0)),
            pl.BlockSpec((1, 1, EH), lambda t, te: (te[t], 0, 0)),
            pl.BlockSpec((1, EH, OUT), lambda t, te: (te[t], 0, 0)),
            pl.BlockSpec((1, 1, 128), lambda t, te: (te[t], 0, 0)),
        ],
        out_specs=pl.BlockSpec((TE, OUT), lambda t, te: (t, 0)),
    )
    y_sorted = pl.pallas_call(
        _expert_kernel,
        grid_spec=grid_spec,
        out_shape=jax.ShapeDtypeStruct((M_PAD, OUT), f32),
        compiler_params=pltpu.CompilerParams(
            dimension_semantics=("arbitrary",)),
    )(tile_expert, rows, bf(exp_W0), exp_b0[:, None, :], bf(exp_W1),
      exp_b1[:, None, :], bf(exp_W2), exp_b2[:, None, :], bf(exp_Wo),
      bo128[:, None, :])

    y1 = y_sorted[pos1]
    y2 = y_sorted[pos2]
    out = pl.pallas_call(
        _combine_kernel,
        grid=(N // TILE,),
        in_specs=[pl.BlockSpec((TILE, 2), lambda i: (i, 0)),
                  pl.BlockSpec((TILE, OUT), lambda i: (i, 0)),
                  pl.BlockSpec((TILE, OUT), lambda i: (i, 0))],
        out_specs=pl.BlockSpec((TILE, OUT), lambda i: (i, 0)),
        out_shape=jax.ShapeDtypeStruct((N, OUT), f32),
        compiler_params=pltpu.CompilerParams(
            dimension_semantics=("arbitrary",)),
    )(g, y1, y2)
    return out
